# baseline (speedup 1.0000x reference)
import functools

import jax
import jax.numpy as jnp
from jax import lax
from jax.experimental import pallas as pl
from jax.experimental.pallas import tpu as pltpu, tpu_sc as plsc

C = 128
V = 10000
M = 320000
NW = 32
RG = 8
MG = NW // RG
M_PER_W = M // MG
ROWS_PER_G = C // RG
IB = 40000
NIB = M_PER_W // IB
S = 10000
NSUB = IB // S
NOBUF = 4


def _sc_gather_concat(t0, t1, idx):
    mesh = plsc.VectorSubcoreMesh(core_axis_name="c", subcore_axis_name="s")

    @functools.partial(
        pl.kernel,
        out_type=jax.ShapeDtypeStruct((2 * C * M,), jnp.float32),
        mesh=mesh,
        scratch_types=[
            pltpu.VMEM((IB,), jnp.int32),
            pltpu.VMEM((V,), jnp.float32),
            pltpu.VMEM((V,), jnp.float32),
            pltpu.VMEM((S,), jnp.float32),
            pltpu.VMEM((S,), jnp.float32),
            pltpu.VMEM((S,), jnp.float32),
            pltpu.VMEM((S,), jnp.float32),
            pltpu.SemaphoreType.DMA,
            pltpu.SemaphoreType.DMA,
            pltpu.SemaphoreType.DMA,
            pltpu.SemaphoreType.DMA,
            pltpu.SemaphoreType.DMA,
            pltpu.SemaphoreType.DMA,
        ],
        compiler_params=pltpu.CompilerParams(needs_layout_passes=False),
    )
    def body(t0_hbm, t1_hbm, idx_hbm, out_hbm,
             idx_v, row_v0, row_v1, ov0, ov1, ov2, ov3,
             rsem0, rsem1, os0, os1, os2, os3):
        wid = lax.axis_index("s") * 2 + lax.axis_index("c")
        rg = wid % RG
        mg = wid // RG
        m_base = mg * M_PER_W
        row0 = rg * ROWS_PER_G

        row_bufs = (row_v0, row_v1)
        row_sems = (rsem0, rsem1)
        out_bufs = (ov0, ov1, ov2, ov3)
        out_sems = (os0, os1, os2, os3)

        def row_src(t_hbm, r):
            return t_hbm.at[pl.ds((row0 + r) * V, V)]

        def out_dst(half, r, ib, s):
            off = ((half * C + row0 + r) * M
                   + m_base + ib * IB + s * S)
            return out_hbm.at[pl.ds(off, S)]

        def gather_sub(s, row_vb, out_vb):
            @plsc.parallel_loop(0, S // 16, unroll=16)
            def _g(j):
                ids = idx_v[pl.ds(s * S + j * 16, 16)]
                out_vb[pl.ds(j * 16, 16)] = plsc.load_gather(row_vb, [ids])

        def do_row(half, t_hbm, ib, r, rb, wait_subs):
            row_vb = row_bufs[rb]
            pltpu.make_async_copy(row_src(t_hbm, r), row_vb,
                                  row_sems[rb]).wait()

            @pl.when(r + 1 < ROWS_PER_G)
            def _():
                pltpu.async_copy(row_src(t_hbm, r + 1), row_bufs[1 - rb],
                                 row_sems[1 - rb])

            for s in range(NSUB):
                ob = s % NOBUF
                if wait_subs:
                    pltpu.make_async_copy(out_bufs[ob],
                                          out_dst(half, r, ib, s),
                                          out_sems[ob]).wait()
                gather_sub(s, row_vb, out_bufs[ob])
                pltpu.async_copy(out_bufs[ob], out_dst(half, r, ib, s),
                                 out_sems[ob])

        first_segment = True
        for half, t_hbm in ((0, t0_hbm), (1, t1_hbm)):
            for ib in range(NIB):
                pltpu.sync_copy(
                    idx_hbm.at[pl.ds(half * M + m_base + ib * IB, IB)],
                    idx_v)
                pltpu.async_copy(row_src(t_hbm, 0), row_v0, rsem0)

                if first_segment:
                    do_row(half, t_hbm, ib, 0, 0, wait_subs=False)
                    do_row(half, t_hbm, ib, 1, 1, wait_subs=True)
                    pair_lo = 1
                    first_segment = False
                else:
                    pair_lo = 0

                def pair_body(rp, _, half=half, t_hbm=t_hbm, ib=ib):
                    for b in (0, 1):
                        do_row(half, t_hbm, ib, rp * 2 + b, b,
                               wait_subs=True)
                    return 0

                lax.fori_loop(pair_lo, ROWS_PER_G // 2, pair_body, 0)

        for ob in range(NOBUF):
            pltpu.make_async_copy(out_bufs[ob],
                                  out_dst(1, ROWS_PER_G - 1, NIB - 1, ob),
                                  out_sems[ob]).wait()

    return body(t0, t1, idx)


def kernel(in0, in1, matches):
    t0 = in0.reshape(C * V)
    t1 = in1.reshape(C * V)
    idx = matches.astype(jnp.int32).reshape(2 * M)
    out = _sc_gather_concat(t0, t1, idx)
    return out.reshape(1, 2 * C, M)


# speedup vs baseline: 1.0025x; 1.0025x over previous
import functools

import jax
import jax.numpy as jnp
from jax import lax
from jax.experimental import pallas as pl
from jax.experimental.pallas import tpu as pltpu, tpu_sc as plsc

C = 128
V = 10000
M = 320000
NW = 32
RG = 8
MG = NW // RG
M_PER_W = M // MG
ROWS_PER_G = C // RG
IB = 40000
NIB = M_PER_W // IB
S = 20000
NSUB = IB // S
NOBUF = 2


def _sc_gather_concat(t0, t1, idx):
    mesh = plsc.VectorSubcoreMesh(core_axis_name="c", subcore_axis_name="s")

    @functools.partial(
        pl.kernel,
        out_type=jax.ShapeDtypeStruct((2 * C * M,), jnp.float32),
        mesh=mesh,
        scratch_types=[
            pltpu.VMEM((IB,), jnp.int32),
            pltpu.VMEM((V,), jnp.float32),
            pltpu.VMEM((V,), jnp.float32),
            pltpu.VMEM((S,), jnp.float32),
            pltpu.VMEM((S,), jnp.float32),
            pltpu.SemaphoreType.DMA,
            pltpu.SemaphoreType.DMA,
            pltpu.SemaphoreType.DMA,
            pltpu.SemaphoreType.DMA,
        ],
        compiler_params=pltpu.CompilerParams(needs_layout_passes=False),
    )
    def body(t0_hbm, t1_hbm, idx_hbm, out_hbm,
             idx_v, row_v0, row_v1, ov0, ov1,
             rsem0, rsem1, os0, os1):
        wid = lax.axis_index("s") * 2 + lax.axis_index("c")
        rg = wid % RG
        mg = wid // RG
        m_base = mg * M_PER_W
        row0 = rg * ROWS_PER_G

        row_bufs = (row_v0, row_v1)
        row_sems = (rsem0, rsem1)
        out_bufs = (ov0, ov1)
        out_sems = (os0, os1)

        def row_src(t_hbm, r):
            return t_hbm.at[pl.ds((row0 + r) * V, V)]

        def out_dst(half, r, ib, s):
            off = ((half * C + row0 + r) * M
                   + m_base + ib * IB + s * S)
            return out_hbm.at[pl.ds(off, S)]

        def gather_sub(s, row_vb, out_vb):
            @plsc.parallel_loop(0, S // 16, unroll=16)
            def _g(j):
                ids = idx_v[pl.ds(s * S + j * 16, 16)]
                out_vb[pl.ds(j * 16, 16)] = plsc.load_gather(row_vb, [ids])

        def do_row(half, t_hbm, ib, r, rb, wait_subs):
            row_vb = row_bufs[rb]
            pltpu.make_async_copy(row_src(t_hbm, r), row_vb,
                                  row_sems[rb]).wait()

            @pl.when(r + 1 < ROWS_PER_G)
            def _():
                pltpu.async_copy(row_src(t_hbm, r + 1), row_bufs[1 - rb],
                                 row_sems[1 - rb])

            for s in range(NSUB):
                ob = s % NOBUF
                if wait_subs:
                    pltpu.make_async_copy(out_bufs[ob],
                                          out_dst(half, r, ib, s),
                                          out_sems[ob]).wait()
                gather_sub(s, row_vb, out_bufs[ob])
                pltpu.async_copy(out_bufs[ob], out_dst(half, r, ib, s),
                                 out_sems[ob])

        first_segment = True
        for half, t_hbm in ((0, t0_hbm), (1, t1_hbm)):
            for ib in range(NIB):
                pltpu.sync_copy(
                    idx_hbm.at[pl.ds(half * M + m_base + ib * IB, IB)],
                    idx_v)
                pltpu.async_copy(row_src(t_hbm, 0), row_v0, rsem0)

                if first_segment:
                    do_row(half, t_hbm, ib, 0, 0, wait_subs=False)
                    do_row(half, t_hbm, ib, 1, 1, wait_subs=True)
                    pair_lo = 1
                    first_segment = False
                else:
                    pair_lo = 0

                def pair_body(rp, _, half=half, t_hbm=t_hbm, ib=ib):
                    for b in (0, 1):
                        do_row(half, t_hbm, ib, rp * 2 + b, b,
                               wait_subs=True)
                    return 0

                lax.fori_loop(pair_lo, ROWS_PER_G // 2, pair_body, 0)

        for ob in range(NOBUF):
            pltpu.make_async_copy(out_bufs[ob],
                                  out_dst(1, ROWS_PER_G - 1, NIB - 1, ob),
                                  out_sems[ob]).wait()

    return body(t0, t1, idx)


def kernel(in0, in1, matches):
    t0 = in0.reshape(C * V)
    t1 = in1.reshape(C * V)
    idx = matches.astype(jnp.int32).reshape(2 * M)
    out = _sc_gather_concat(t0, t1, idx)
    return out.reshape(1, 2 * C, M)


# speedup vs baseline: 1.0778x; 1.0750x over previous
import functools

import jax
import jax.numpy as jnp
from jax import lax
from jax.experimental import pallas as pl
from jax.experimental.pallas import tpu as pltpu, tpu_sc as plsc

C = 128
V = 10000
M = 320000
NW = 32
RG = 8
MG = NW // RG
M_PER_W = M // MG
ROWS_PER_G = C // RG
NPAIR = ROWS_PER_G // 2
IB = 40000
NIB = M_PER_W // IB
S = 10000
NSUB = IB // S


def _sc_gather_concat(t0, t1, idx):
    mesh = plsc.VectorSubcoreMesh(core_axis_name="c", subcore_axis_name="s")

    @functools.partial(
        pl.kernel,
        out_type=jax.ShapeDtypeStruct((2 * C * M,), jnp.float32),
        mesh=mesh,
        scratch_types=[
            pltpu.VMEM((IB,), jnp.int32),
            pltpu.VMEM((V,), jnp.float32),
            pltpu.VMEM((V,), jnp.float32),
            pltpu.VMEM((V,), jnp.float32),
            pltpu.VMEM((V,), jnp.float32),
            pltpu.VMEM((S,), jnp.float32),
            pltpu.VMEM((S,), jnp.float32),
            pltpu.VMEM((S,), jnp.float32),
            pltpu.VMEM((S,), jnp.float32),
            pltpu.SemaphoreType.DMA,
            pltpu.SemaphoreType.DMA,
            pltpu.SemaphoreType.DMA,
            pltpu.SemaphoreType.DMA,
            pltpu.SemaphoreType.DMA,
            pltpu.SemaphoreType.DMA,
        ],
        compiler_params=pltpu.CompilerParams(needs_layout_passes=False),
    )
    def body(t0_hbm, t1_hbm, idx_hbm, out_hbm,
             idx_v, rv0a, rv0b, rv1a, rv1b, ov0a, ov0b, ov1a, ov1b,
             rsem0, rsem1, os0a, os0b, os1a, os1b):
        wid = lax.axis_index("s") * 2 + lax.axis_index("c")
        rg = wid % RG
        mg = wid // RG
        m_base = mg * M_PER_W
        row0 = rg * ROWS_PER_G

        row_bufs = ((rv0a, rv0b), (rv1a, rv1b))
        row_sems = (rsem0, rsem1)
        out_bufs = ((ov0a, ov0b), (ov1a, ov1b))
        out_sems = ((os0a, os0b), (os1a, os1b))

        def row_src(t_hbm, r):
            return t_hbm.at[pl.ds((row0 + r) * V, V)]

        def out_dst(half, r, ib, s):
            off = ((half * C + row0 + r) * M
                   + m_base + ib * IB + s * S)
            return out_hbm.at[pl.ds(off, S)]

        def gather_pair(s, rows, outs):
            ra, rb = rows
            oa, ob_ = outs

            @plsc.parallel_loop(0, S // 16, unroll=8)
            def _g(j):
                ids = idx_v[pl.ds(s * S + j * 16, 16)]
                sl = pl.ds(j * 16, 16)
                oa[sl] = plsc.load_gather(ra, [ids])
                ob_[sl] = plsc.load_gather(rb, [ids])

        def prefetch_pair(t_hbm, pr, pb):
            for k in (0, 1):
                pltpu.async_copy(row_src(t_hbm, pr * 2 + k),
                                 row_bufs[pb][k], row_sems[pb])

        def do_pair(half, t_hbm, ib, pr, pb, wait_subs):
            rows = row_bufs[pb]
            for k in (0, 1):
                pltpu.make_async_copy(row_src(t_hbm, pr * 2 + k), rows[k],
                                      row_sems[pb]).wait()

            @pl.when(pr + 1 < NPAIR)
            def _():
                prefetch_pair(t_hbm, pr + 1, 1 - pb)

            for s in range(NSUB):
                ose = s % 2
                if wait_subs or s >= 2:
                    for k in (0, 1):
                        pltpu.make_async_copy(
                            out_bufs[ose][k],
                            out_dst(half, pr * 2 + k, ib, s),
                            out_sems[ose][k]).wait()
                gather_pair(s, rows, out_bufs[ose])
                for k in (0, 1):
                    pltpu.async_copy(out_bufs[ose][k],
                                     out_dst(half, pr * 2 + k, ib, s),
                                     out_sems[ose][k])

        first_segment = True
        for half, t_hbm in ((0, t0_hbm), (1, t1_hbm)):
            for ib in range(NIB):
                pltpu.sync_copy(
                    idx_hbm.at[pl.ds(half * M + m_base + ib * IB, IB)],
                    idx_v)
                prefetch_pair(t_hbm, 0, 0)

                if first_segment:
                    do_pair(half, t_hbm, ib, 0, 0, wait_subs=False)
                    do_pair(half, t_hbm, ib, 1, 1, wait_subs=True)
                    pp_lo = 1
                    first_segment = False
                else:
                    pp_lo = 0

                def pp_body(pp, _, half=half, t_hbm=t_hbm, ib=ib):
                    for b in (0, 1):
                        do_pair(half, t_hbm, ib, pp * 2 + b, b,
                                wait_subs=True)
                    return 0

                lax.fori_loop(pp_lo, NPAIR // 2, pp_body, 0)

        for ose in (0, 1):
            for k in (0, 1):
                pltpu.make_async_copy(
                    out_bufs[ose][k],
                    out_dst(1, ROWS_PER_G - 2 + k, NIB - 1, 2 + ose),
                    out_sems[ose][k]).wait()

    return body(t0, t1, idx)


def kernel(in0, in1, matches):
    t0 = in0.reshape(C * V)
    t1 = in1.reshape(C * V)
    idx = matches.astype(jnp.int32).reshape(2 * M)
    out = _sc_gather_concat(t0, t1, idx)
    return out.reshape(1, 2 * C, M)
